# baseline (speedup 1.0000x reference)
import jax
import jax.numpy as jnp
from jax import lax
from jax.experimental import pallas as pl
from jax.experimental.pallas import tpu as pltpu
from jax.experimental.pallas import tpu_sc as plsc

NV = 100000
NF = 200000
L = 16
NC, NS = 2, 16
NW = NC * NS

CH = 128
FW = 6272
FW3 = 3 * FW
NCH = FW3 // CH
NSTEP = FW // L
NSTEP3 = FW3 // L
VB = 3136
LAG = 8

_EDGE = ((0, 1), (1, 2), (2, 0))

_MESH = plsc.VectorSubcoreMesh(
    core_axis_name="c", subcore_axis_name="s",
    num_cores=NC, num_subcores=NS)
_PARAMS = pltpu.CompilerParams(
    needs_layout_passes=False, use_tc_tiling_on_sc=False)


def _wid():
    return lax.axis_index("s") * NC + lax.axis_index("c")


TCB = 1000
TCG = NF // TCB


def _ofaces_tc_body(f_ref, o_ref):
    b = pl.program_id(1)
    k = pl.program_id(0)
    f = f_ref[...]
    x0, x1, x2 = f[:, 0:1], f[:, 1:2], f[:, 2:3]
    c0 = jnp.where(b == 0, x0, jnp.where(b == 1, x1, x2))
    c1 = jnp.where(b == 0, x1, jnp.where(b == 1, x2, x0))
    rows = (lax.broadcasted_iota(jnp.int32, (TCB, 1), 0) + k * TCB + NV)
    o_ref[...] = jnp.concatenate([c0, c1, rows], axis=1)


def _centroid_body(verts_hbm, faces_hbm, overts_hbm, e0, e1, e2, g0, g1, g2,
                   sem):
    fbase = jnp.minimum(_wid() * FW, NF - FW)
    vbase = jnp.minimum(_wid() * VB, NV - VB)
    iota = lax.iota(jnp.int32, L)

    pltpu.sync_copy(verts_hbm.at[pl.ds(3 * vbase, 3 * VB)],
                    g0.at[pl.ds(0, 3 * VB)])
    pltpu.sync_copy(g0.at[pl.ds(0, 3 * VB)],
                    overts_hbm.at[pl.ds(3 * vbase, 3 * VB)])

    pltpu.sync_copy(faces_hbm.at[pl.ds(3 * fbase, FW3)], e2)

    def _build(c, dst):
        def step(i, _):
            kv = iota + i * L
            jv = kv // 3
            dv = kv - jv * 3
            f = plsc.load_gather(e2, [kv - dv + c])
            dst[pl.ds(i * L, L)] = f * 3 + dv
            return 0
        lax.fori_loop(0, NSTEP3, step, 0)

    _build(0, e0)
    _build(1, e1)
    _build(2, e2)

    def _gather(m, _):
        @pl.when(m < NCH)
        def _():
            for eb, gb in ((e0, g0), (e1, g1), (e2, g2)):
                off = m * CH
                pltpu.async_copy(
                    verts_hbm.at[eb.at[pl.ds(off, CH)]],
                    gb.at[pl.ds(off, CH)], sem)

        @pl.when(m >= LAG)
        def _():
            for eb, gb in ((e0, g0), (e1, g1), (e2, g2)):
                pltpu.make_async_copy(
                    verts_hbm.at[eb.at[pl.ds(0, CH)]],
                    gb.at[pl.ds(0, CH)], sem).wait()
        return 0

    lax.fori_loop(0, NCH + LAG, _gather, 0)

    def _mean(i, _):
        s = pl.ds(i * L, L)
        g0[s] = (g0[s] + g1[s] + g2[s]) / 3.0
        return 0

    lax.fori_loop(0, NSTEP3, _mean, 0)
    pltpu.sync_copy(g0, overts_hbm.at[pl.ds(3 * (NV + fbase), FW3)])


@jax.jit
def _upsample(verts_flat, faces_flat, faces2d):
    faces_fn = pl.pallas_call(
        _ofaces_tc_body,
        grid=(TCG, 3),
        in_specs=[pl.BlockSpec((TCB, 3), lambda k, b: (k, 0))],
        out_specs=pl.BlockSpec((TCB, 3), lambda k, b: (b * TCG + k, 0)),
        out_shape=jax.ShapeDtypeStruct((3 * NF, 3), jnp.int32),
    )
    cen_fn = pl.kernel(
        _centroid_body,
        out_type=jax.ShapeDtypeStruct((3 * (NV + NF),), jnp.float32),
        mesh=_MESH,
        compiler_params=_PARAMS,
        scratch_types=[
            pltpu.VMEM((FW3,), jnp.int32),
            pltpu.VMEM((FW3,), jnp.int32),
            pltpu.VMEM((FW3,), jnp.int32),
            pltpu.VMEM((FW3,), jnp.float32),
            pltpu.VMEM((FW3,), jnp.float32),
            pltpu.VMEM((FW3,), jnp.float32),
            pltpu.SemaphoreType.DMA,
        ],
    )
    overts_flat = cen_fn(verts_flat, faces_flat)
    ofaces = faces_fn(faces2d)
    return overts_flat.reshape(NV + NF, 3), ofaces


def kernel(vertices, faces):
    faces_i32 = faces.astype(jnp.int32)
    return _upsample(vertices.reshape(-1), faces_i32.reshape(-1), faces_i32)


# speedup vs baseline: 2.0217x; 2.0217x over previous
import jax
import jax.numpy as jnp
from jax import lax
from jax.experimental import pallas as pl
from jax.experimental.pallas import tpu as pltpu
from jax.experimental.pallas import tpu_sc as plsc

NV = 100000
NF = 200000
L = 16
NC, NS = 2, 16
NW = NC * NS

CH = 128
FW = 6272
FW3 = 3 * FW
NCH = FW3 // CH
NSTEP = FW // L
NSTEP3 = FW3 // L
VB = 3136
LAG = 8

_EDGE = ((0, 1), (1, 2), (2, 0))

_MESH = plsc.VectorSubcoreMesh(
    core_axis_name="c", subcore_axis_name="s",
    num_cores=NC, num_subcores=NS)
_PARAMS = pltpu.CompilerParams(
    needs_layout_passes=False, use_tc_tiling_on_sc=False)


def _wid():
    return lax.axis_index("s") * NC + lax.axis_index("c")


def _faces_body(faces_hbm, ofaces_hbm, fbuf, c0buf, c1buf, c2buf, ibuf, sem):
    fbase = jnp.minimum(_wid() * FW, NF - FW)
    iota = lax.iota(jnp.int32, L)

    pltpu.sync_copy(faces_hbm.at[pl.ds(3 * fbase, FW3)], fbuf)

    def _extract(i, _):
        j = iota + i * L
        j3 = j * 3
        for c, col in ((0, c0buf), (1, c1buf), (2, c2buf)):
            col[pl.ds(i * L, L)] = plsc.load_gather(fbuf, [j3 + c])
        ibuf[pl.ds(i * L, L)] = NV + fbase + j
        return 0

    lax.fori_loop(0, NSTEP, _extract, 0)

    cols = (c0buf, c1buf, c2buf)
    for b, (p0, p1) in enumerate(_EDGE):
        base = b * NF + fbase
        pltpu.sync_copy(cols[p0], ofaces_hbm.at[pl.ds(base, FW)])
        pltpu.sync_copy(cols[p1], ofaces_hbm.at[pl.ds(3 * NF + base, FW)])
        pltpu.sync_copy(ibuf, ofaces_hbm.at[pl.ds(6 * NF + base, FW)])


def _centroid_body(verts_hbm, faces_hbm, overts_hbm, e0, e1, e2, g0, g1, g2,
                   sem):
    fbase = jnp.minimum(_wid() * FW, NF - FW)
    vbase = jnp.minimum(_wid() * VB, NV - VB)
    iota = lax.iota(jnp.int32, L)

    pltpu.sync_copy(verts_hbm.at[pl.ds(3 * vbase, 3 * VB)],
                    g0.at[pl.ds(0, 3 * VB)])
    pltpu.sync_copy(g0.at[pl.ds(0, 3 * VB)],
                    overts_hbm.at[pl.ds(3 * vbase, 3 * VB)])

    pltpu.sync_copy(faces_hbm.at[pl.ds(3 * fbase, FW3)], e2)

    def _build(c, dst):
        def step(i, _):
            kv = iota + i * L
            jv = kv // 3
            dv = kv - jv * 3
            f = plsc.load_gather(e2, [kv - dv + c])
            dst[pl.ds(i * L, L)] = f * 3 + dv
            return 0
        lax.fori_loop(0, NSTEP3, step, 0)

    _build(0, e0)
    _build(1, e1)
    _build(2, e2)

    def _gather(m, _):
        @pl.when(m < NCH)
        def _():
            for eb, gb in ((e0, g0), (e1, g1), (e2, g2)):
                off = m * CH
                pltpu.async_copy(
                    verts_hbm.at[eb.at[pl.ds(off, CH)]],
                    gb.at[pl.ds(off, CH)], sem)

        @pl.when(m >= LAG)
        def _():
            for eb, gb in ((e0, g0), (e1, g1), (e2, g2)):
                pltpu.make_async_copy(
                    verts_hbm.at[eb.at[pl.ds(0, CH)]],
                    gb.at[pl.ds(0, CH)], sem).wait()
        return 0

    lax.fori_loop(0, NCH + LAG, _gather, 0)

    def _mean(i, _):
        s = pl.ds(i * L, L)
        g0[s] = (g0[s] + g1[s] + g2[s]) / 3.0
        return 0

    lax.fori_loop(0, NSTEP3, _mean, 0)
    pltpu.sync_copy(g0, overts_hbm.at[pl.ds(3 * (NV + fbase), FW3)])


@jax.jit
def _upsample(verts_flat, faces_flat):
    faces_fn = pl.kernel(
        _faces_body,
        out_type=jax.ShapeDtypeStruct((3 * NF * 3,), jnp.int32),
        mesh=_MESH,
        compiler_params=_PARAMS,
        scratch_types=[
            pltpu.VMEM((FW3,), jnp.int32),
            pltpu.VMEM((FW,), jnp.int32),
            pltpu.VMEM((FW,), jnp.int32),
            pltpu.VMEM((FW,), jnp.int32),
            pltpu.VMEM((FW,), jnp.int32),
            pltpu.SemaphoreType.DMA,
        ],
    )
    cen_fn = pl.kernel(
        _centroid_body,
        out_type=jax.ShapeDtypeStruct((3 * (NV + NF),), jnp.float32),
        mesh=_MESH,
        compiler_params=_PARAMS,
        scratch_types=[
            pltpu.VMEM((FW3,), jnp.int32),
            pltpu.VMEM((FW3,), jnp.int32),
            pltpu.VMEM((FW3,), jnp.int32),
            pltpu.VMEM((FW3,), jnp.float32),
            pltpu.VMEM((FW3,), jnp.float32),
            pltpu.VMEM((FW3,), jnp.float32),
            pltpu.SemaphoreType.DMA,
        ],
    )
    ofaces_col = faces_fn(faces_flat)
    overts_flat = cen_fn(verts_flat, faces_flat)
    ofaces = ofaces_col.reshape(3, 3 * NF).T
    return overts_flat.reshape(NV + NF, 3), ofaces


def kernel(vertices, faces):
    return _upsample(vertices.reshape(-1),
                     faces.astype(jnp.int32).reshape(-1))


# speedup vs baseline: 3.2363x; 1.6008x over previous
import jax
import jax.numpy as jnp
from jax import lax
from jax.experimental import pallas as pl
from jax.experimental.pallas import tpu as pltpu
from jax.experimental.pallas import tpu_sc as plsc

NV = 100000
NF = 200000
L = 16
NC, NS = 2, 16
NW = NC * NS

CH = 128
FW = 6272
FW3 = 3 * FW
NCH = FW3 // CH
NSTEP = FW // L
NSTEP3 = FW3 // L
VB = 3136
LAG = 8

_EDGE = ((0, 1), (1, 2), (2, 0))

_MESH = plsc.VectorSubcoreMesh(
    core_axis_name="c", subcore_axis_name="s",
    num_cores=NC, num_subcores=NS)
_PARAMS = pltpu.CompilerParams(
    needs_layout_passes=False, use_tc_tiling_on_sc=False)


def _wid():
    return lax.axis_index("s") * NC + lax.axis_index("c")


def _faces_body(faces_hbm, ofaces_hbm, fbuf, c0buf, c1buf, c2buf, ibuf, sem):
    fbase = jnp.minimum(_wid() * FW, NF - FW)
    iota = lax.iota(jnp.int32, L)

    pltpu.sync_copy(faces_hbm.at[pl.ds(3 * fbase, FW3)], fbuf)

    def _extract(i, _):
        j = iota + i * L
        j3 = j * 3
        for c, col in ((0, c0buf), (1, c1buf), (2, c2buf)):
            col[pl.ds(i * L, L)] = plsc.load_gather(fbuf, [j3 + c])
        ibuf[pl.ds(i * L, L)] = NV + fbase + j
        return 0

    lax.fori_loop(0, NSTEP, _extract, 0)

    cols = (c0buf, c1buf, c2buf)
    for b, (p0, p1) in enumerate(_EDGE):
        base = b * NF + fbase
        pltpu.sync_copy(cols[p0], ofaces_hbm.at[pl.ds(base, FW)])
        pltpu.sync_copy(cols[p1], ofaces_hbm.at[pl.ds(3 * NF + base, FW)])
        pltpu.sync_copy(ibuf, ofaces_hbm.at[pl.ds(6 * NF + base, FW)])


def _centroid_body(verts_hbm, faces_hbm, overts_hbm, e0, e1, e2, g0, g1, g2,
                   sem):
    fbase = jnp.minimum(_wid() * FW, NF - FW)
    vbase = jnp.minimum(_wid() * VB, NV - VB)
    iota = lax.iota(jnp.int32, L)
    NO = NV + NF

    pltpu.sync_copy(verts_hbm.at[pl.ds(3 * vbase, 3 * VB)],
                    g0.at[pl.ds(0, 3 * VB)])

    def _vcols(i, _):
        j = iota + i * L
        j3 = j * 3
        for d in range(3):
            g1[pl.ds(d * VB + i * L, L)] = plsc.load_gather(g0, [j3 + d])
        return 0

    lax.fori_loop(0, VB // L, _vcols, 0)
    for d in range(3):
        pltpu.sync_copy(g1.at[pl.ds(d * VB, VB)],
                        overts_hbm.at[pl.ds(d * NO + vbase, VB)])

    pltpu.sync_copy(faces_hbm.at[pl.ds(3 * fbase, FW3)], e2)

    def _build(c, dst):
        def step(i, _):
            kv = iota + i * L
            jv = kv // 3
            dv = kv - jv * 3
            f = plsc.load_gather(e2, [kv - dv + c])
            dst[pl.ds(i * L, L)] = f * 3 + dv
            return 0
        lax.fori_loop(0, NSTEP3, step, 0)

    _build(0, e0)
    _build(1, e1)
    _build(2, e2)

    def _gather(m, _):
        @pl.when(m < NCH)
        def _():
            for eb, gb in ((e0, g0), (e1, g1), (e2, g2)):
                off = m * CH
                pltpu.async_copy(
                    verts_hbm.at[eb.at[pl.ds(off, CH)]],
                    gb.at[pl.ds(off, CH)], sem)

        @pl.when(m >= LAG)
        def _():
            for eb, gb in ((e0, g0), (e1, g1), (e2, g2)):
                pltpu.make_async_copy(
                    verts_hbm.at[eb.at[pl.ds(0, CH)]],
                    gb.at[pl.ds(0, CH)], sem).wait()
        return 0

    lax.fori_loop(0, NCH + LAG, _gather, 0)

    def _mean(i, _):
        s = pl.ds(i * L, L)
        g0[s] = (g0[s] + g1[s] + g2[s]) / 3.0
        return 0

    lax.fori_loop(0, NSTEP3, _mean, 0)

    def _ccols(i, _):
        j = iota + i * L
        j3 = j * 3
        for d in range(3):
            g1[pl.ds(d * FW + i * L, L)] = plsc.load_gather(g0, [j3 + d])
        return 0

    lax.fori_loop(0, NSTEP, _ccols, 0)
    for d in range(3):
        pltpu.sync_copy(g1.at[pl.ds(d * FW, FW)],
                        overts_hbm.at[pl.ds(d * NO + NV + fbase, FW)])


@jax.jit
def _upsample(verts_flat, faces_flat):
    faces_fn = pl.kernel(
        _faces_body,
        out_type=jax.ShapeDtypeStruct((3 * NF * 3,), jnp.int32),
        mesh=_MESH,
        compiler_params=_PARAMS,
        scratch_types=[
            pltpu.VMEM((FW3,), jnp.int32),
            pltpu.VMEM((FW,), jnp.int32),
            pltpu.VMEM((FW,), jnp.int32),
            pltpu.VMEM((FW,), jnp.int32),
            pltpu.VMEM((FW,), jnp.int32),
            pltpu.SemaphoreType.DMA,
        ],
    )
    cen_fn = pl.kernel(
        _centroid_body,
        out_type=jax.ShapeDtypeStruct((3 * (NV + NF),), jnp.float32),
        mesh=_MESH,
        compiler_params=_PARAMS,
        scratch_types=[
            pltpu.VMEM((FW3,), jnp.int32),
            pltpu.VMEM((FW3,), jnp.int32),
            pltpu.VMEM((FW3,), jnp.int32),
            pltpu.VMEM((FW3,), jnp.float32),
            pltpu.VMEM((FW3,), jnp.float32),
            pltpu.VMEM((FW3,), jnp.float32),
            pltpu.SemaphoreType.DMA,
        ],
    )
    ofaces_col = faces_fn(faces_flat)
    overts_col = cen_fn(verts_flat, faces_flat)
    ofaces = ofaces_col.reshape(3, 3 * NF).T
    overts = overts_col.reshape(3, NV + NF).T
    return overts, ofaces


def kernel(vertices, faces):
    return _upsample(vertices.reshape(-1),
                     faces.astype(jnp.int32).reshape(-1))


# speedup vs baseline: 7.5659x; 2.3378x over previous
import jax
import jax.numpy as jnp
from jax import lax
from jax.experimental import pallas as pl
from jax.experimental.pallas import tpu as pltpu
from jax.experimental.pallas import tpu_sc as plsc

NV = 100000
NF = 200000
NO = NV + NF
L = 16
NC, NS = 2, 16
NW = NC * NS

CH = 128
FW = 6272
NCH = FW // CH
NSTEP = FW // L
VB = 3136
LAG = 8

_EDGE = ((0, 1), (1, 2), (2, 0))

_MESH = plsc.VectorSubcoreMesh(
    core_axis_name="c", subcore_axis_name="s",
    num_cores=NC, num_subcores=NS)
_PARAMS = pltpu.CompilerParams(
    needs_layout_passes=False, use_tc_tiling_on_sc=False)


def _wid():
    return lax.axis_index("s") * NC + lax.axis_index("c")


def _faces_body(faces_hbm, ofaces_hbm, c0, c1, c2, ibuf, sem):
    fbase = jnp.minimum(_wid() * FW, NF - FW)
    iota = lax.iota(jnp.int32, L)

    cols = (c0, c1, c2)
    for c in range(3):
        pltpu.sync_copy(faces_hbm.at[pl.ds(c * NF + fbase, FW)], cols[c])

    def _itail(i, _):
        ibuf[pl.ds(i * L, L)] = NV + fbase + iota + i * L
        return 0

    lax.fori_loop(0, NSTEP, _itail, 0)

    for b, (p0, p1) in enumerate(_EDGE):
        base = b * NF + fbase
        pltpu.sync_copy(cols[p0], ofaces_hbm.at[pl.ds(base, FW)])
        pltpu.sync_copy(cols[p1], ofaces_hbm.at[pl.ds(3 * NF + base, FW)])
        pltpu.sync_copy(ibuf, ofaces_hbm.at[pl.ds(6 * NF + base, FW)])


def _centroid_body(verts_hbm, faces_hbm, overts_hbm, c0, c1, c2, g0, g1, g2,
                   sem):
    fbase = jnp.minimum(_wid() * FW, NF - FW)
    vbase = jnp.minimum(_wid() * VB, NV - VB)
    iota = lax.iota(jnp.int32, L)

    for d in range(3):
        pltpu.sync_copy(verts_hbm.at[pl.ds(d * NV + vbase, VB)],
                        g0.at[pl.ds(0, VB)])
        pltpu.sync_copy(g0.at[pl.ds(0, VB)],
                        overts_hbm.at[pl.ds(d * NO + vbase, VB)])

    cols = (c0, c1, c2)
    for c in range(3):
        pltpu.sync_copy(faces_hbm.at[pl.ds(c * NF + fbase, FW)], cols[c])

    def _bump(i, _):
        s = pl.ds(i * L, L)
        c0[s] = c0[s] + NV
        c1[s] = c1[s] + NV
        c2[s] = c2[s] + NV
        return 0

    def _mean(i, _):
        s = pl.ds(i * L, L)
        g0[s] = (g0[s] + g1[s] + g2[s]) / 3.0
        return 0

    def _gather(m, _):
        @pl.when(m < NCH)
        def _():
            for cb, gb in ((c0, g0), (c1, g1), (c2, g2)):
                off = m * CH
                pltpu.async_copy(
                    verts_hbm.at[cb.at[pl.ds(off, CH)]],
                    gb.at[pl.ds(off, CH)], sem)

        @pl.when(m >= LAG)
        def _():
            for cb, gb in ((c0, g0), (c1, g1), (c2, g2)):
                pltpu.make_async_copy(
                    verts_hbm.at[cb.at[pl.ds(0, CH)]],
                    gb.at[pl.ds(0, CH)], sem).wait()
        return 0

    for d in range(3):
        if d:
            lax.fori_loop(0, NSTEP, _bump, 0)
        lax.fori_loop(0, NCH + LAG, _gather, 0)
        lax.fori_loop(0, NSTEP, _mean, 0)
        pltpu.sync_copy(g0, overts_hbm.at[pl.ds(d * NO + NV + fbase, FW)])


@jax.jit
def _upsample(verts_col, faces_col):
    faces_fn = pl.kernel(
        _faces_body,
        out_type=jax.ShapeDtypeStruct((3 * NF * 3,), jnp.int32),
        mesh=_MESH,
        compiler_params=_PARAMS,
        scratch_types=[
            pltpu.VMEM((FW,), jnp.int32),
            pltpu.VMEM((FW,), jnp.int32),
            pltpu.VMEM((FW,), jnp.int32),
            pltpu.VMEM((FW,), jnp.int32),
            pltpu.SemaphoreType.DMA,
        ],
    )
    cen_fn = pl.kernel(
        _centroid_body,
        out_type=jax.ShapeDtypeStruct((3 * NO,), jnp.float32),
        mesh=_MESH,
        compiler_params=_PARAMS,
        scratch_types=[
            pltpu.VMEM((FW,), jnp.int32),
            pltpu.VMEM((FW,), jnp.int32),
            pltpu.VMEM((FW,), jnp.int32),
            pltpu.VMEM((FW,), jnp.float32),
            pltpu.VMEM((FW,), jnp.float32),
            pltpu.VMEM((FW,), jnp.float32),
            pltpu.SemaphoreType.DMA,
        ],
    )
    ofaces_col = faces_fn(faces_col)
    overts_col = cen_fn(verts_col, faces_col)
    ofaces = ofaces_col.reshape(3, 3 * NF).T
    overts = overts_col.reshape(3, NO).T
    return overts, ofaces


def kernel(vertices, faces):
    return _upsample(vertices.T.reshape(-1),
                     faces.astype(jnp.int32).T.reshape(-1))
